# Initial kernel scaffold; baseline (speedup 1.0000x reference)
import jax
import jax.numpy as jnp
from jax.experimental import pallas as pl


def kernel(input, edge_index, W, b):
    raise NotImplementedError("write your pallas kernel here")


# speedup vs baseline: 5.2111x; 5.2111x over previous
import functools

import jax
import jax.numpy as jnp
from jax import lax
from jax.experimental import pallas as pl
from jax.experimental.pallas import tpu as pltpu
from jax.experimental.pallas import tpu_sc as plsc

N_NODES = 10000
N_EDGES = 320000
F = 128

NC = 2
NS = 16
NW = NC * NS
CHUNK = 128
N_PAD = 10240
ROWS_PER_TILE = N_PAD // NS
CHUNKS_PER_TILE = -(-N_EDGES // (NW * CHUNK))
E_PAD = NW * CHUNKS_PER_TILE * CHUNK


def _sc_aggregate_body(x_hbm, dst_hbm, src_hbm, agg_out, deg_out,
                       didx_all, sidx_all, rows, degv, agg_sp, sem):
    c = lax.axis_index("c")
    s = lax.axis_index("s")
    wid = c * NS + s

    zero16 = jnp.zeros((16,), jnp.float32)
    one16 = jnp.full((16,), 1.0, jnp.float32)

    def _zrow(i, _):
        for g in range(F // 16):
            rows[i, pl.ds(g * 16, 16)] = zero16
        return 0
    lax.fori_loop(0, CHUNK, _zrow, 0)

    def _zdeg(k, _):
        degv[pl.ds(k * 16, 16)] = zero16
        return 0
    lax.fori_loop(0, N_PAD // 16, _zdeg, 0)

    for t in range(ROWS_PER_TILE // CHUNK):
        pltpu.sync_copy(rows, agg_sp.at[pl.ds(s * ROWS_PER_TILE + t * CHUNK, CHUNK)])

    pltpu.sync_copy(dst_hbm.at[wid], didx_all)
    pltpu.sync_copy(src_hbm.at[wid], sidx_all)

    plsc.subcore_barrier()

    def _edge_chunk(j, _):
        pltpu.async_copy(x_hbm.at[didx_all.at[j]], rows, sem).wait()
        pltpu.sync_copy(rows, agg_sp.at[sidx_all.at[j]], add=True)
        for g in range(CHUNK // 16):
            idx16 = sidx_all[j, pl.ds(g * 16, 16)]
            plsc.addupdate_scatter(degv, [idx16], one16)
        return 0

    lax.fori_loop(0, CHUNKS_PER_TILE, _edge_chunk, 0)

    plsc.subcore_barrier()

    pltpu.sync_copy(agg_sp.at[pl.ds(s * ROWS_PER_TILE, ROWS_PER_TILE)],
                    agg_out.at[c, pl.ds(s * ROWS_PER_TILE, ROWS_PER_TILE)])
    pltpu.sync_copy(degv, deg_out.at[wid])


_sc_aggregate = pl.kernel(
    _sc_aggregate_body,
    out_type=(
        jax.ShapeDtypeStruct((NC, N_PAD, F), jnp.float32),
        jax.ShapeDtypeStruct((NW, N_PAD), jnp.float32),
    ),
    mesh=plsc.VectorSubcoreMesh(core_axis_name="c", subcore_axis_name="s"),
    compiler_params=pltpu.CompilerParams(needs_layout_passes=False),
    scratch_types=[
        pltpu.VMEM((CHUNKS_PER_TILE, CHUNK), jnp.int32),
        pltpu.VMEM((CHUNKS_PER_TILE, CHUNK), jnp.int32),
        pltpu.VMEM((CHUNK, F), jnp.float32),
        pltpu.VMEM((N_PAD,), jnp.float32),
        pltpu.VMEM_SHARED((N_PAD, F), jnp.float32),
        pltpu.SemaphoreType.DMA,
    ],
)


def _tc_finish_body(x_ref, agg_ref, deg_ref, w_ref, b_ref, out_ref):
    h = agg_ref[0] + agg_ref[1] + x_ref[...]
    support = jax.lax.dot_general(
        h, w_ref[...], (((1,), (0,)), ((), ())),
        preferred_element_type=jnp.float32,
        precision=jax.lax.Precision.HIGHEST)
    deg = jnp.sum(deg_ref[...], axis=0) + 1.0
    out_ref[...] = support / deg[:, None] + b_ref[...]


def _tc_finish(x_pad, agg, deg, W, b2d):
    blk = 1024
    grid = (N_PAD // blk,)
    return pl.pallas_call(
        _tc_finish_body,
        grid=grid,
        in_specs=[
            pl.BlockSpec((blk, F), lambda i: (i, 0)),
            pl.BlockSpec((NC, blk, F), lambda i: (0, i, 0)),
            pl.BlockSpec((NW, blk), lambda i: (0, i)),
            pl.BlockSpec((F, F), lambda i: (0, 0)),
            pl.BlockSpec((1, F), lambda i: (0, 0)),
        ],
        out_specs=pl.BlockSpec((blk, F), lambda i: (i, 0)),
        out_shape=jax.ShapeDtypeStruct((N_PAD, F), jnp.float32),
    )(x_pad, agg, deg, W, b2d)


def kernel(input, edge_index, W, b):
    x_pad = jnp.zeros((N_PAD, F), jnp.float32).at[:N_NODES].set(input)
    ei = edge_index.astype(jnp.int32)
    pad = E_PAD - N_EDGES
    dst = jnp.concatenate([ei[1], jnp.full((pad,), N_NODES, jnp.int32)])
    src = jnp.concatenate([ei[0], jnp.full((pad,), N_NODES, jnp.int32)])
    dst_r = dst.reshape(NW, CHUNKS_PER_TILE, CHUNK)
    src_r = src.reshape(NW, CHUNKS_PER_TILE, CHUNK)

    agg, deg = _sc_aggregate(x_pad, dst_r, src_r)
    out = _tc_finish(x_pad, agg, deg, W, b.reshape(1, F))
    return out[:N_NODES]


# speedup vs baseline: 9.4706x; 1.8174x over previous
import functools

import jax
import jax.numpy as jnp
from jax import lax
from jax.experimental import pallas as pl
from jax.experimental.pallas import tpu as pltpu
from jax.experimental.pallas import tpu_sc as plsc

N_NODES = 10000
N_EDGES = 320000
F = 128

NC = 2
NS = 16
NW = NC * NS
CHUNK = 128
META = 40
N_META = 2
N_PAD = 10240
ROWS_PER_TILE = N_PAD // NS
E_PAD = NW * N_META * META * CHUNK


def _sc_aggregate_body(x_hbm, dst_hbm, src_hbm, agg_out, deg_out,
                       didx, sidx, rows, degv, agg_sp, gsem):
    c = lax.axis_index("c")
    s = lax.axis_index("s")
    wid = c * NS + s

    zero16 = jnp.zeros((16,), jnp.float32)
    one16 = jnp.full((16,), 1.0, jnp.float32)

    _ns = jax.named_scope
    def _zrow(i, _):
        for g in range(F // 16):
            rows[i, pl.ds(g * 16, 16)] = zero16
        return 0
    with _ns("zero_rows"):
        lax.fori_loop(0, CHUNK, _zrow, 0)

    def _zdeg(k, _):
        degv[pl.ds(k * 16, 16)] = zero16
        return 0
    with _ns("zero_deg"):
        lax.fori_loop(0, N_PAD // 16, _zdeg, 0)

    def _zslab(t, _):
        pltpu.sync_copy(rows, agg_sp.at[pl.ds(s * ROWS_PER_TILE + t * CHUNK, CHUNK)])
        return 0
    with _ns("zero_slab"):
        lax.fori_loop(0, ROWS_PER_TILE // CHUNK, _zslab, 0)

    with _ns("barrier1"):
        plsc.subcore_barrier()

    def _meta(m, _):
        off = wid * (N_META * META) + m * META
        pltpu.sync_copy(dst_hbm.at[pl.ds(off, META)], didx)
        pltpu.sync_copy(src_hbm.at[pl.ds(off, META)], sidx)

        def _chunk(j, _):
            pltpu.async_copy(x_hbm.at[didx.at[j]], rows, gsem).wait()
            sc = pltpu.async_copy(rows, agg_sp.at[sidx.at[j]], gsem, add=True)
            for g in range(CHUNK // 16):
                idx16 = sidx[j, pl.ds(g * 16, 16)]
                plsc.addupdate_scatter(degv, [idx16], one16)
            sc.wait()
            return 0

        lax.fori_loop(0, META, _chunk, 0)
        return 0

    with _ns("edges"):
        lax.fori_loop(0, N_META, _meta, 0)

    with _ns("barrier2"):
        plsc.subcore_barrier()

    with _ns("drain"):
        pltpu.sync_copy(agg_sp.at[pl.ds(s * ROWS_PER_TILE, ROWS_PER_TILE)],
                        agg_out.at[c, pl.ds(s * ROWS_PER_TILE, ROWS_PER_TILE)])
        pltpu.sync_copy(degv, deg_out.at[wid])


_sc_aggregate = pl.kernel(
    _sc_aggregate_body,
    out_type=(
        jax.ShapeDtypeStruct((NC, N_PAD, F), jnp.float32),
        jax.ShapeDtypeStruct((NW, N_PAD), jnp.float32),
    ),
    mesh=plsc.VectorSubcoreMesh(core_axis_name="c", subcore_axis_name="s"),
    compiler_params=pltpu.CompilerParams(needs_layout_passes=False),
    scratch_types=[
        pltpu.VMEM((META, CHUNK), jnp.int32),
        pltpu.VMEM((META, CHUNK), jnp.int32),
        pltpu.VMEM((CHUNK, F), jnp.float32),
        pltpu.VMEM((N_PAD,), jnp.float32),
        pltpu.VMEM_SHARED((N_PAD, F), jnp.float32),
        pltpu.SemaphoreType.DMA,
    ],
)


def _tc_finish_body(x_ref, agg_ref, deg_ref, w_ref, b_ref, out_ref):
    h = agg_ref[0] + agg_ref[1] + x_ref[...]
    support = jax.lax.dot_general(
        h, w_ref[...], (((1,), (0,)), ((), ())),
        preferred_element_type=jnp.float32,
        precision=jax.lax.Precision.HIGHEST)
    deg = jnp.sum(deg_ref[...], axis=0) + 1.0
    out_ref[...] = support / deg[:, None] + b_ref[...]


def _tc_finish(x, agg, deg, W, b2d):
    blk = 1024
    grid = (N_PAD // blk,)
    return pl.pallas_call(
        _tc_finish_body,
        grid=grid,
        in_specs=[
            pl.BlockSpec((blk, F), lambda i: (i, 0)),
            pl.BlockSpec((NC, blk, F), lambda i: (0, i, 0)),
            pl.BlockSpec((NW, blk), lambda i: (0, i)),
            pl.BlockSpec((F, F), lambda i: (0, 0)),
            pl.BlockSpec((1, F), lambda i: (0, 0)),
        ],
        out_specs=pl.BlockSpec((blk, F), lambda i: (i, 0)),
        out_shape=jax.ShapeDtypeStruct((N_NODES, F), jnp.float32),
    )(x, agg, deg, W, b2d)


def kernel(input, edge_index, W, b):
    ei = edge_index.astype(jnp.int32)
    pad = E_PAD - N_EDGES
    dst = jnp.concatenate([ei[1], jnp.arange(pad, dtype=jnp.int32) % N_NODES])
    src = jnp.concatenate(
        [ei[0], N_NODES + (jnp.arange(pad, dtype=jnp.int32) % (N_PAD - N_NODES))])
    dst_r = dst.reshape(E_PAD // CHUNK, CHUNK)
    src_r = src.reshape(E_PAD // CHUNK, CHUNK)

    agg, deg = _sc_aggregate(input, dst_r, src_r)
    return _tc_finish(input, agg, deg, W, b.reshape(1, F))


# speedup vs baseline: 11.9972x; 1.2668x over previous
import functools

import jax
import jax.numpy as jnp
from jax import lax
from jax.experimental import pallas as pl
from jax.experimental.pallas import tpu as pltpu
from jax.experimental.pallas import tpu_sc as plsc

N_NODES = 10000
N_EDGES = 320000
F = 128

NC = 2
NS = 16
NW = NC * NS
CHUNK = 128
META = 8
N_META = 10
N_PAD = 10240
ROWS_PER_TILE = N_PAD // NS
E_PAD = NW * N_META * META * CHUNK


def _sc_aggregate_body(x_hbm, dst_hbm, src_hbm, agg_out, deg_out,
                       didx, sidx, rows_a, rows_b, degv, agg_sp,
                       gsem_a, gsem_b, ssem_a, ssem_b):
    rows = rows_a
    c = lax.axis_index("c")
    s = lax.axis_index("s")
    wid = c * NS + s

    zero16 = jnp.zeros((16,), jnp.float32)
    one16 = jnp.full((16,), 1.0, jnp.float32)

    _ns = jax.named_scope
    def _zrow(i, _):
        for g in range(F // 16):
            rows[i, pl.ds(g * 16, 16)] = zero16
        return 0
    with _ns("zero_rows"):
        lax.fori_loop(0, CHUNK, _zrow, 0)

    def _zdeg(k, _):
        degv[pl.ds(k * 16, 16)] = zero16
        return 0
    with _ns("zero_deg"):
        lax.fori_loop(0, N_PAD // 16, _zdeg, 0)

    def _zslab(t, _):
        pltpu.sync_copy(rows, agg_sp.at[pl.ds(s * ROWS_PER_TILE + t * CHUNK, CHUNK)])
        return 0
    with _ns("zero_slab"):
        lax.fori_loop(0, ROWS_PER_TILE // CHUNK, _zslab, 0)

    with _ns("barrier1"):
        plsc.subcore_barrier()

    def _deg_adds(j):
        for g in range(CHUNK // 16):
            idx16 = sidx[j, pl.ds(g * 16, 16)]
            plsc.addupdate_scatter(degv, [idx16], one16)

    def _meta(m, _):
        off = wid * (N_META * META) + m * META
        pltpu.sync_copy(dst_hbm.at[pl.ds(off, META)], didx)
        pltpu.sync_copy(src_hbm.at[pl.ds(off, META)], sidx)

        pltpu.async_copy(x_hbm.at[didx.at[0]], rows_a, gsem_a)
        pltpu.async_copy(x_hbm.at[didx.at[1]], rows_b, gsem_b)

        def _pair(i, _):
            j = i * 2
            pltpu.make_async_copy(x_hbm.at[didx.at[j]], rows_a, gsem_a).wait()
            sc_a = pltpu.async_copy(rows_a, agg_sp.at[sidx.at[j]], ssem_a, add=True)
            _deg_adds(j)
            sc_a.wait()

            @pl.when(j + 2 < META)
            def _():
                pltpu.async_copy(x_hbm.at[didx.at[j + 2]], rows_a, gsem_a)

            pltpu.make_async_copy(x_hbm.at[didx.at[j + 1]], rows_b, gsem_b).wait()
            sc_b = pltpu.async_copy(rows_b, agg_sp.at[sidx.at[j + 1]], ssem_b, add=True)
            _deg_adds(j + 1)
            sc_b.wait()

            @pl.when(j + 3 < META)
            def _():
                pltpu.async_copy(x_hbm.at[didx.at[j + 3]], rows_b, gsem_b)
            return 0

        lax.fori_loop(0, META // 2, _pair, 0)
        return 0

    with _ns("edges"):
        lax.fori_loop(0, N_META, _meta, 0)

    with _ns("barrier2"):
        plsc.subcore_barrier()

    with _ns("drain"):
        pltpu.sync_copy(agg_sp.at[pl.ds(s * ROWS_PER_TILE, ROWS_PER_TILE)],
                        agg_out.at[c, pl.ds(s * ROWS_PER_TILE, ROWS_PER_TILE)])
        pltpu.sync_copy(degv, deg_out.at[wid])


_sc_aggregate = pl.kernel(
    _sc_aggregate_body,
    out_type=(
        jax.ShapeDtypeStruct((NC, N_PAD, F), jnp.float32),
        jax.ShapeDtypeStruct((NW, N_PAD), jnp.float32),
    ),
    mesh=plsc.VectorSubcoreMesh(core_axis_name="c", subcore_axis_name="s"),
    compiler_params=pltpu.CompilerParams(needs_layout_passes=False),
    scratch_types=[
        pltpu.VMEM((META, CHUNK), jnp.int32),
        pltpu.VMEM((META, CHUNK), jnp.int32),
        pltpu.VMEM((CHUNK, F), jnp.float32),
        pltpu.VMEM((CHUNK, F), jnp.float32),
        pltpu.VMEM((N_PAD,), jnp.float32),
        pltpu.VMEM_SHARED((N_PAD, F), jnp.float32),
        pltpu.SemaphoreType.DMA,
        pltpu.SemaphoreType.DMA,
        pltpu.SemaphoreType.DMA,
        pltpu.SemaphoreType.DMA,
    ],
)


def _tc_finish_body(x_ref, agg_ref, deg_ref, w_ref, b_ref, out_ref):
    h = agg_ref[0] + agg_ref[1] + x_ref[...]
    support = jax.lax.dot_general(
        h, w_ref[...], (((1,), (0,)), ((), ())),
        preferred_element_type=jnp.float32,
        precision=jax.lax.Precision.HIGHEST)
    deg = jnp.sum(deg_ref[...], axis=0) + 1.0
    out_ref[...] = support / deg[:, None] + b_ref[...]


def _tc_finish(x, agg, deg, W, b2d):
    blk = 1024
    grid = (N_PAD // blk,)
    return pl.pallas_call(
        _tc_finish_body,
        grid=grid,
        in_specs=[
            pl.BlockSpec((blk, F), lambda i: (i, 0)),
            pl.BlockSpec((NC, blk, F), lambda i: (0, i, 0)),
            pl.BlockSpec((NW, blk), lambda i: (0, i)),
            pl.BlockSpec((F, F), lambda i: (0, 0)),
            pl.BlockSpec((1, F), lambda i: (0, 0)),
        ],
        out_specs=pl.BlockSpec((blk, F), lambda i: (i, 0)),
        out_shape=jax.ShapeDtypeStruct((N_NODES, F), jnp.float32),
    )(x, agg, deg, W, b2d)


def kernel(input, edge_index, W, b):
    ei = edge_index.astype(jnp.int32)
    pad = E_PAD - N_EDGES
    dst = jnp.concatenate([ei[1], jnp.arange(pad, dtype=jnp.int32) % N_NODES])
    src = jnp.concatenate(
        [ei[0], N_NODES + (jnp.arange(pad, dtype=jnp.int32) % (N_PAD - N_NODES))])
    dst_r = dst.reshape(E_PAD // CHUNK, CHUNK)
    src_r = src.reshape(E_PAD // CHUNK, CHUNK)

    agg, deg = _sc_aggregate(input, dst_r, src_r)
    return _tc_finish(input, agg, deg, W, b.reshape(1, F))


# speedup vs baseline: 12.3987x; 1.0335x over previous
import functools

import jax
import jax.numpy as jnp
from jax import lax
from jax.experimental import pallas as pl
from jax.experimental.pallas import tpu as pltpu
from jax.experimental.pallas import tpu_sc as plsc

N_NODES = 10000
N_EDGES = 320000
F = 128

NC = 2
NS = 16
NW = NC * NS
CHUNK = 128
META = 8
N_META = 10
N_PAD = 10240
ROWS_PER_TILE = N_PAD // NS
E_PAD = NW * N_META * META * CHUNK
E_ROWS = N_EDGES // CHUNK
TAIL_ROWS = 64
TAIL_ROW0 = E_PAD // CHUNK - TAIL_ROWS


def _sc_aggregate_body(x_hbm, dst_hbm, src_hbm, tdst_hbm, tsrc_hbm,
                       agg_out, deg_out,
                       didx, sidx, rows_a, rows_b, degv, agg_sp,
                       gsem_a, gsem_b, ssem_a, ssem_b, isem):
    rows = rows_a
    c = lax.axis_index("c")
    s = lax.axis_index("s")
    wid = c * NS + s

    zero16 = jnp.zeros((16,), jnp.float32)
    one16 = jnp.full((16,), 1.0, jnp.float32)

    _ns = jax.named_scope
    def _zrow(i, _):
        for g in range(F // 16):
            rows[i, pl.ds(g * 16, 16)] = zero16
        return 0
    with _ns("zero_rows"):
        lax.fori_loop(0, CHUNK, _zrow, 0)

    def _zdeg(k, _):
        degv[pl.ds(k * 16, 16)] = zero16
        return 0
    with _ns("zero_deg"):
        lax.fori_loop(0, N_PAD // 16, _zdeg, 0)

    def _zslab(t, _):
        pltpu.sync_copy(rows, agg_sp.at[pl.ds(s * ROWS_PER_TILE + t * CHUNK, CHUNK)])
        return 0
    with _ns("zero_slab"):
        lax.fori_loop(0, ROWS_PER_TILE // CHUNK, _zslab, 0)

    with _ns("barrier1"):
        plsc.subcore_barrier()

    def _deg_adds(j):
        for g in range(CHUNK // 16):
            idx16 = sidx[j, pl.ds(g * 16, 16)]
            plsc.addupdate_scatter(degv, [idx16], one16)

    def _meta(m, _):
        off = wid * (N_META * META) + m * META
        in_tail = off >= TAIL_ROW0

        @pl.when(in_tail)
        def _():
            toff = off - TAIL_ROW0
            sidx_cp = pltpu.async_copy(tsrc_hbm.at[pl.ds(toff, META)], sidx, isem)
            pltpu.sync_copy(tdst_hbm.at[pl.ds(toff, META)], didx)
            sidx_cp.wait()

        @pl.when(jnp.logical_not(in_tail))
        def _():
            sidx_cp = pltpu.async_copy(src_hbm.at[pl.ds(off, META)], sidx, isem)
            pltpu.sync_copy(dst_hbm.at[pl.ds(off, META)], didx)
            sidx_cp.wait()

        pltpu.async_copy(x_hbm.at[didx.at[0]], rows_a, gsem_a)
        pltpu.async_copy(x_hbm.at[didx.at[1]], rows_b, gsem_b)

        def _pair(i, _):
            j = i * 2
            pltpu.make_async_copy(x_hbm.at[didx.at[j]], rows_a, gsem_a).wait()
            sc_a = pltpu.async_copy(rows_a, agg_sp.at[sidx.at[j]], ssem_a, add=True)
            _deg_adds(j)
            sc_a.wait()

            @pl.when(j + 2 < META)
            def _():
                pltpu.async_copy(x_hbm.at[didx.at[j + 2]], rows_a, gsem_a)

            pltpu.make_async_copy(x_hbm.at[didx.at[j + 1]], rows_b, gsem_b).wait()
            sc_b = pltpu.async_copy(rows_b, agg_sp.at[sidx.at[j + 1]], ssem_b, add=True)
            _deg_adds(j + 1)
            sc_b.wait()

            @pl.when(j + 3 < META)
            def _():
                pltpu.async_copy(x_hbm.at[didx.at[j + 3]], rows_b, gsem_b)
            return 0

        lax.fori_loop(0, META // 2, _pair, 0)
        return 0

    with _ns("edges"):
        lax.fori_loop(0, N_META, _meta, 0)

    with _ns("barrier2"):
        plsc.subcore_barrier()

    with _ns("drain"):
        pltpu.sync_copy(agg_sp.at[pl.ds(s * ROWS_PER_TILE, ROWS_PER_TILE)],
                        agg_out.at[c, pl.ds(s * ROWS_PER_TILE, ROWS_PER_TILE)])
        pltpu.sync_copy(degv, deg_out.at[wid])


_sc_aggregate = pl.kernel(
    _sc_aggregate_body,
    out_type=(
        jax.ShapeDtypeStruct((NC, N_PAD, F), jnp.float32),
        jax.ShapeDtypeStruct((NW, N_PAD), jnp.float32),
    ),
    mesh=plsc.VectorSubcoreMesh(core_axis_name="c", subcore_axis_name="s"),
    compiler_params=pltpu.CompilerParams(needs_layout_passes=False),
    scratch_types=[
        pltpu.VMEM((META, CHUNK), jnp.int32),
        pltpu.VMEM((META, CHUNK), jnp.int32),
        pltpu.VMEM((CHUNK, F), jnp.float32),
        pltpu.VMEM((CHUNK, F), jnp.float32),
        pltpu.VMEM((N_PAD,), jnp.float32),
        pltpu.VMEM_SHARED((N_PAD, F), jnp.float32),
        pltpu.SemaphoreType.DMA,
        pltpu.SemaphoreType.DMA,
        pltpu.SemaphoreType.DMA,
        pltpu.SemaphoreType.DMA,
        pltpu.SemaphoreType.DMA,
    ],
)


def _tc_finish_body(x_ref, agg_ref, deg_ref, w_ref, b_ref, out_ref):
    h = agg_ref[0] + agg_ref[1] + x_ref[...]
    support = jax.lax.dot_general(
        h, w_ref[...], (((1,), (0,)), ((), ())),
        preferred_element_type=jnp.float32,
        precision=jax.lax.Precision.HIGHEST)
    deg = jnp.sum(deg_ref[...], axis=0) + 1.0
    out_ref[...] = support / deg[:, None] + b_ref[...]


def _tc_finish(x, agg, deg, W, b2d):
    blk = 1024
    grid = (N_PAD // blk,)
    return pl.pallas_call(
        _tc_finish_body,
        grid=grid,
        in_specs=[
            pl.BlockSpec((blk, F), lambda i: (i, 0)),
            pl.BlockSpec((NC, blk, F), lambda i: (0, i, 0)),
            pl.BlockSpec((NW, blk), lambda i: (0, i)),
            pl.BlockSpec((F, F), lambda i: (0, 0)),
            pl.BlockSpec((1, F), lambda i: (0, 0)),
        ],
        out_specs=pl.BlockSpec((blk, F), lambda i: (i, 0)),
        out_shape=jax.ShapeDtypeStruct((N_NODES, F), jnp.float32),
    )(x, agg, deg, W, b2d)


def kernel(input, edge_index, W, b):
    ei = edge_index.astype(jnp.int32)
    pad = E_PAD - N_EDGES
    dst_r = ei[1].reshape(E_ROWS, CHUNK)
    src_r = ei[0].reshape(E_ROWS, CHUNK)
    tail0 = TAIL_ROW0 * CHUNK
    dum_dst = (jnp.arange(pad, dtype=jnp.int32) % N_NODES).reshape(-1, CHUNK)
    dum_src = (N_NODES + jnp.arange(pad, dtype=jnp.int32)
               % (N_PAD - N_NODES)).reshape(-1, CHUNK)
    tdst = jnp.concatenate([dst_r[TAIL_ROW0:], dum_dst])
    tsrc = jnp.concatenate([src_r[TAIL_ROW0:], dum_src])

    agg, deg = _sc_aggregate(input, dst_r, src_r, tdst, tsrc)
    return _tc_finish(input, agg, deg, W, b.reshape(1, F))
